# baseline (speedup 1.0000x reference)
import numpy as np

import jax
import jax.numpy as jnp
from jax.experimental import pallas as pl
from jax.experimental.pallas import tpu as pltpu


def _matmul_body(x1_ref, x2_ref, p_ref, o_ref):
    p = p_ref[0]
    a = jnp.dot(x1_ref[...], p, preferred_element_type=jnp.float32)
    b = jnp.dot(x2_ref[...], p, preferred_element_type=jnp.float32)
    o_ref[...] = jnp.concatenate([a, b], axis=1)[None]


def _make_sort_body(n, n_pad, cc, blk):
    rows = n_pad // 2
    log_n = n_pad.bit_length() - 1
    s_swap = rows.bit_length() - 1
    log_b = blk.bit_length() - 1
    n_blocks = rows // blk
    hi_real = n - rows
    pad_rows = rows - hi_real
    half = cc // 2
    n_small = min(5, log_b)

    def body(z_ref, o_ref, x_ref, sem):
        c = pl.program_id(0)
        cp = pltpu.make_async_copy(z_ref.at[c], x_ref, sem)
        cp.start()
        cp.wait()
        if pad_rows > 0:
            x_ref[pl.ds(hi_real, pad_rows), pl.ds(cc, cc)] = jnp.full(
                (pad_rows, cc), jnp.inf, dtype=jnp.float32)

        r_iota = jax.lax.broadcasted_iota(jnp.int32, (blk, 1), 0)
        lane = jax.lax.broadcasted_iota(jnp.int32, (1, 2 * cc), 1)
        lane_hi = jnp.where(lane >= cc, jnp.int32(rows), jnp.int32(0))
        bs_tab = [(r_iota >> s) & 1 for s in range(n_small)]

        def prologue_block(g, _):
            xb = x_ref[pl.ds(g * blk, blk), :]
            base_r = g * blk + r_iota
            for k in range(1, n_small):
                bk = (base_r >> k) & 1
                for s in range(k - 1, -1, -1):
                    dd = 1 << s
                    up = pltpu.roll(xb, blk - dd, axis=0)
                    dn = pltpu.roll(xb, dd, axis=0)
                    bs = bs_tab[s]
                    p = jnp.where(bs == 0, up, dn)
                    take = (xb > p) == (bs == bk)
                    xb = jnp.where(take, p, xb)
            x_ref[pl.ds(g * blk, blk), :] = xb
            return 0

        jax.lax.fori_loop(0, n_blocks, prologue_block, 0)

        def lane_swap_block(g, _):
            xb = x_ref[pl.ds(g * blk, blk), :]
            lo = xb[:, :cc]
            hi = xb[:, cc:]
            x_ref[pl.ds(g * blk, blk), :] = jnp.concatenate(
                [jnp.minimum(lo, hi), jnp.maximum(lo, hi)], axis=1)
            return 0

        def emit_phase(k, static):
            if static and k == log_n:
                jax.lax.fori_loop(0, n_blocks, lane_swap_block, 0)

            def asc_of(base):
                if static:
                    return (((base + lane_hi) >> k) & 1) == 0
                return ((base >> k) & 1) == 0

            for s in range(s_swap - 1, log_b - 1, -1):
                d = 1 << s
                db = d >> log_b

                def pair(g, _, d=d, db=db):
                    base = ((g // db) * 2 * db + g % db) * blk
                    a = x_ref[pl.ds(base, blk), :]
                    b = x_ref[pl.ds(base + d, blk), :]
                    mn = jnp.minimum(a, b)
                    mx = jnp.maximum(a, b)
                    asc = asc_of(base)
                    x_ref[pl.ds(base, blk), :] = jnp.where(asc, mn, mx)
                    x_ref[pl.ds(base + d, blk), :] = jnp.where(asc, mx, mn)
                    return 0

                if static:
                    jax.lax.fori_loop(0, n_blocks // 2, pair, 0)
                else:
                    @pl.when(k > s)
                    def _(pair=pair):
                        jax.lax.fori_loop(0, n_blocks // 2, pair, 0)

            for s in range(log_b - 1, n_small - 1, -1):
                d = 1 << s
                trips = rows // (2 * d)
                unroll = 2 if trips % 2 == 0 else 1

                def one_pair(base, d=d):
                    a = x_ref[pl.ds(base, d), :]
                    b = x_ref[pl.ds(base + d, d), :]
                    mn = jnp.minimum(a, b)
                    mx = jnp.maximum(a, b)
                    asc = asc_of(base)
                    x_ref[pl.ds(base, d), :] = jnp.where(asc, mn, mx)
                    x_ref[pl.ds(base + d, d), :] = jnp.where(asc, mx, mn)

                def pairm(i, _, d=d, unroll=unroll):
                    for u in range(unroll):
                        one_pair((i * unroll + u) * 2 * d)
                    return 0

                if static:
                    jax.lax.fori_loop(0, trips // unroll, pairm, 0)
                else:
                    @pl.when(k > s)
                    def _(pairm=pairm, trips=trips, unroll=unroll):
                        jax.lax.fori_loop(0, trips // unroll, pairm, 0)

            def small_block(g, _):
                xb = x_ref[pl.ds(g * blk, blk), :]
                if static:
                    bk = ((g * blk + r_iota + lane_hi) >> k) & 1
                else:
                    bk = ((g * blk + r_iota) >> k) & 1
                for s in range(n_small - 1, -1, -1):
                    dd = 1 << s
                    up = pltpu.roll(xb, blk - dd, axis=0)
                    dn = pltpu.roll(xb, dd, axis=0)
                    bs = bs_tab[s]
                    p = jnp.where(bs == 0, up, dn)
                    wm = bs == bk
                    take = (xb > p) == wm
                    xb = jnp.where(take, p, xb)
                x_ref[pl.ds(g * blk, blk), :] = xb
                return 0

            jax.lax.fori_loop(0, n_blocks, small_block, 0)

        def phase(k, _):
            emit_phase(k, static=False)
            return 0

        if n_small < s_swap:
            jax.lax.fori_loop(n_small, s_swap, phase, 0)
        for k in range(max(s_swap, n_small), log_n + 1):
            emit_phase(k, static=True)

        def red_block(g, acc):
            xb = x_ref[pl.ds(g * blk, blk), :]
            dl = xb[:, 0:half] - xb[:, half:cc]
            dh = xb[:, cc:cc + half] - xb[:, cc + half:]
            valid = (r_iota + g * blk) < hi_real
            dh = jnp.where(valid, dh, 0.0)
            return acc + jnp.sum(dl * dl) + jnp.sum(dh * dh)

        acc = jax.lax.fori_loop(0, n_blocks, red_block, jnp.float32(0.0))

        @pl.when(c == 0)
        def _():
            o_ref[...] = jnp.zeros((1, 1), jnp.float32)

        o_ref[...] += acc.reshape(1, 1)

    return body


def kernel(XP, XQ, projections):
    n, f = XP.shape
    l = projections.shape[1]
    n_pad = 1 << (n - 1).bit_length()
    rows = n_pad // 2

    cc = min(64, 2 * l)
    half = cc // 2
    n_chunks = (2 * l) // cc
    blk = min(8192, rows)

    P2 = jnp.zeros((2 * f, 2 * l), dtype=jnp.float32)
    P2 = P2.at[:f, :l].set(projections).at[f:, l:].set(projections)
    perm = []
    for g in range(n_chunks):
        perm += list(range(g * half, (g + 1) * half))
        perm += list(range(l + g * half, l + (g + 1) * half))
    P2 = P2[:, np.array(perm)]
    P3 = P2.reshape(2 * f, n_chunks, cc).transpose(1, 0, 2)

    X2 = jnp.zeros((n_pad, 2 * f), dtype=jnp.float32)
    X2 = X2.at[:n, :f].set(XP).at[:n, f:].set(XQ)

    rb = min(4096, rows)
    n_rb = rows // rb
    Z = pl.pallas_call(
        _matmul_body,
        grid=(n_rb, n_chunks),
        in_specs=[
            pl.BlockSpec((rb, 2 * f), lambda r, c: (r, 0)),
            pl.BlockSpec((rb, 2 * f), lambda r, c, n_rb=n_rb: (r + n_rb, 0)),
            pl.BlockSpec((1, 2 * f, cc), lambda r, c: (c, 0, 0)),
        ],
        out_specs=pl.BlockSpec((1, rb, 2 * cc), lambda r, c: (c, r, 0)),
        out_shape=jax.ShapeDtypeStruct((n_chunks, rows, 2 * cc), jnp.float32),
    )(X2, X2, P3)

    total = pl.pallas_call(
        _make_sort_body(n, n_pad, cc, blk),
        grid=(n_chunks,),
        in_specs=[pl.BlockSpec(memory_space=pltpu.MemorySpace.HBM)],
        out_specs=pl.BlockSpec((1, 1), lambda c: (0, 0)),
        out_shape=jax.ShapeDtypeStruct((1, 1), jnp.float32),
        scratch_shapes=[
            pltpu.VMEM((rows, 2 * cc), jnp.float32),
            pltpu.SemaphoreType.DMA,
        ],
        compiler_params=pltpu.CompilerParams(
            vmem_limit_bytes=67000000),
    )(Z)

    return jnp.sqrt(total[0, 0] / (n * l))


# speedup vs baseline: 1.0008x; 1.0008x over previous
import numpy as np

import jax
import jax.numpy as jnp
from jax.experimental import pallas as pl
from jax.experimental.pallas import tpu as pltpu


def _matmul_body(x1_ref, x2_ref, p_ref, o_ref):
    p = p_ref[0]
    a = jnp.dot(x1_ref[...], p, preferred_element_type=jnp.float32)
    b = jnp.dot(x2_ref[...], p, preferred_element_type=jnp.float32)
    o_ref[...] = jnp.concatenate([a, b], axis=1)[None]


def _make_sort_body(n, n_pad, cc, blk):
    rows = n_pad // 2
    log_n = n_pad.bit_length() - 1
    s_swap = rows.bit_length() - 1
    log_b = blk.bit_length() - 1
    n_blocks = rows // blk
    hi_real = n - rows
    pad_rows = rows - hi_real
    half = cc // 2
    n_small = min(5, log_b)

    def body(z_ref, o_ref, x_ref, sem):
        c = pl.program_id(0)
        cp = pltpu.make_async_copy(z_ref.at[c], x_ref, sem)
        cp.start()
        cp.wait()
        if pad_rows > 0:
            x_ref[pl.ds(hi_real, pad_rows), pl.ds(cc, cc)] = jnp.full(
                (pad_rows, cc), jnp.inf, dtype=jnp.float32)

        r_iota = jax.lax.broadcasted_iota(jnp.int32, (blk, 1), 0)
        lane = jax.lax.broadcasted_iota(jnp.int32, (1, 2 * cc), 1)
        lane_hi = jnp.where(lane >= cc, jnp.int32(rows), jnp.int32(0))
        bs_tab = [(r_iota >> s) & 1 for s in range(n_small)]

        def prologue_block(g, _):
            xb = x_ref[pl.ds(g * blk, blk), :]
            base_r = g * blk + r_iota
            for k in range(1, n_small):
                bk = (base_r >> k) & 1
                for s in range(k - 1, -1, -1):
                    dd = 1 << s
                    up = pltpu.roll(xb, blk - dd, axis=0)
                    dn = pltpu.roll(xb, dd, axis=0)
                    bs = bs_tab[s]
                    p = jnp.where(bs == 0, up, dn)
                    take = (xb > p) == (bs == bk)
                    xb = jnp.where(take, p, xb)
            x_ref[pl.ds(g * blk, blk), :] = xb
            return 0

        jax.lax.fori_loop(0, n_blocks, prologue_block, 0)

        def lane_swap_block(g, _):
            xb = x_ref[pl.ds(g * blk, blk), :]
            lo = xb[:, :cc]
            hi = xb[:, cc:]
            x_ref[pl.ds(g * blk, blk), :] = jnp.concatenate(
                [jnp.minimum(lo, hi), jnp.maximum(lo, hi)], axis=1)
            return 0

        def emit_phase(k, static):
            if static and k == log_n:
                jax.lax.fori_loop(0, n_blocks, lane_swap_block, 0)

            def asc_of(base):
                if static:
                    return (((base + lane_hi) >> k) & 1) == 0
                return ((base >> k) & 1) == 0

            for s in range(s_swap - 1, log_b - 1, -1):
                d = 1 << s
                db = d >> log_b

                def pair(g, _, d=d, db=db):
                    base = ((g // db) * 2 * db + g % db) * blk
                    a = x_ref[pl.ds(base, blk), :]
                    b = x_ref[pl.ds(base + d, blk), :]
                    mn = jnp.minimum(a, b)
                    mx = jnp.maximum(a, b)
                    asc = asc_of(base)
                    x_ref[pl.ds(base, blk), :] = jnp.where(asc, mn, mx)
                    x_ref[pl.ds(base + d, blk), :] = jnp.where(asc, mx, mn)
                    return 0

                if static:
                    jax.lax.fori_loop(0, n_blocks // 2, pair, 0)
                else:
                    @pl.when(k > s)
                    def _(pair=pair):
                        jax.lax.fori_loop(0, n_blocks // 2, pair, 0)

            for s in range(log_b - 1, n_small - 1, -1):
                d = 1 << s
                trips = rows // (2 * d)
                unroll = 2 if trips % 2 == 0 else 1

                def one_pair(base, d=d):
                    a = x_ref[pl.ds(base, d), :]
                    b = x_ref[pl.ds(base + d, d), :]
                    mn = jnp.minimum(a, b)
                    mx = jnp.maximum(a, b)
                    asc = asc_of(base)
                    x_ref[pl.ds(base, d), :] = jnp.where(asc, mn, mx)
                    x_ref[pl.ds(base + d, d), :] = jnp.where(asc, mx, mn)

                def pairm(i, _, d=d, unroll=unroll):
                    for u in range(unroll):
                        one_pair((i * unroll + u) * 2 * d)
                    return 0

                if static:
                    jax.lax.fori_loop(0, trips // unroll, pairm, 0)
                else:
                    @pl.when(k > s)
                    def _(pairm=pairm, trips=trips, unroll=unroll):
                        jax.lax.fori_loop(0, trips // unroll, pairm, 0)

            def small_block(g, _):
                xb = x_ref[pl.ds(g * blk, blk), :]
                if static:
                    bk = ((g * blk + r_iota + lane_hi) >> k) & 1
                else:
                    bk = ((g * blk + r_iota) >> k) & 1
                for s in range(n_small - 1, -1, -1):
                    dd = 1 << s
                    up = pltpu.roll(xb, blk - dd, axis=0)
                    dn = pltpu.roll(xb, dd, axis=0)
                    bs = bs_tab[s]
                    p = jnp.where(bs == 0, up, dn)
                    wm = bs == bk
                    take = (xb > p) == wm
                    xb = jnp.where(take, p, xb)
                x_ref[pl.ds(g * blk, blk), :] = xb
                return 0

            jax.lax.fori_loop(0, n_blocks, small_block, 0)

        def phase(k, _):
            emit_phase(k, static=False)
            return 0

        if n_small < s_swap:
            jax.lax.fori_loop(n_small, s_swap, phase, 0)
        for k in range(max(s_swap, n_small), log_n + 1):
            emit_phase(k, static=True)

        def red_block(g, acc):
            xb = x_ref[pl.ds(g * blk, blk), :]
            dl = xb[:, 0:half] - xb[:, half:cc]
            dh = xb[:, cc:cc + half] - xb[:, cc + half:]
            valid = (r_iota + g * blk) < hi_real
            dh = jnp.where(valid, dh, 0.0)
            return acc + jnp.sum(dl * dl) + jnp.sum(dh * dh)

        acc = jax.lax.fori_loop(0, n_blocks, red_block, jnp.float32(0.0))

        @pl.when(c == 0)
        def _():
            o_ref[...] = jnp.zeros((1, 1), jnp.float32)

        o_ref[...] += acc.reshape(1, 1)

    return body


def kernel(XP, XQ, projections):
    n, f = XP.shape
    l = projections.shape[1]
    n_pad = 1 << (n - 1).bit_length()
    rows = n_pad // 2

    cc = min(64, 2 * l)
    half = cc // 2
    n_chunks = (2 * l) // cc
    blk = min(4096, rows)

    P2 = jnp.zeros((2 * f, 2 * l), dtype=jnp.float32)
    P2 = P2.at[:f, :l].set(projections).at[f:, l:].set(projections)
    perm = []
    for g in range(n_chunks):
        perm += list(range(g * half, (g + 1) * half))
        perm += list(range(l + g * half, l + (g + 1) * half))
    P2 = P2[:, np.array(perm)]
    P3 = P2.reshape(2 * f, n_chunks, cc).transpose(1, 0, 2)

    X2 = jnp.zeros((n_pad, 2 * f), dtype=jnp.float32)
    X2 = X2.at[:n, :f].set(XP).at[:n, f:].set(XQ)

    rb = min(4096, rows)
    n_rb = rows // rb
    Z = pl.pallas_call(
        _matmul_body,
        grid=(n_rb, n_chunks),
        in_specs=[
            pl.BlockSpec((rb, 2 * f), lambda r, c: (r, 0)),
            pl.BlockSpec((rb, 2 * f), lambda r, c, n_rb=n_rb: (r + n_rb, 0)),
            pl.BlockSpec((1, 2 * f, cc), lambda r, c: (c, 0, 0)),
        ],
        out_specs=pl.BlockSpec((1, rb, 2 * cc), lambda r, c: (c, r, 0)),
        out_shape=jax.ShapeDtypeStruct((n_chunks, rows, 2 * cc), jnp.float32),
    )(X2, X2, P3)

    total = pl.pallas_call(
        _make_sort_body(n, n_pad, cc, blk),
        grid=(n_chunks,),
        in_specs=[pl.BlockSpec(memory_space=pltpu.MemorySpace.HBM)],
        out_specs=pl.BlockSpec((1, 1), lambda c: (0, 0)),
        out_shape=jax.ShapeDtypeStruct((1, 1), jnp.float32),
        scratch_shapes=[
            pltpu.VMEM((rows, 2 * cc), jnp.float32),
            pltpu.SemaphoreType.DMA,
        ],
    )(Z)

    return jnp.sqrt(total[0, 0] / (n * l))


# speedup vs baseline: 1.0224x; 1.0216x over previous
import numpy as np

import jax
import jax.numpy as jnp
from jax.experimental import pallas as pl
from jax.experimental.pallas import tpu as pltpu


def _matmul_body(x1_ref, x2_ref, p_ref, o_ref):
    p = p_ref[0]
    a = jnp.dot(x1_ref[...], p, preferred_element_type=jnp.float32)
    b = jnp.dot(x2_ref[...], p, preferred_element_type=jnp.float32)
    o_ref[...] = jnp.concatenate([a, b], axis=1)[None]


def _make_sort_body(n, n_pad, cc, blk):
    rows = n_pad // 2
    log_n = n_pad.bit_length() - 1
    s_swap = rows.bit_length() - 1
    log_b = blk.bit_length() - 1
    n_blocks = rows // blk
    hi_real = n - rows
    pad_rows = rows - hi_real
    half = cc // 2
    n_small = min(5, log_b)

    def body(z_ref, o_ref, x_ref, sem):
        c = pl.program_id(0)
        cp = pltpu.make_async_copy(z_ref.at[c], x_ref, sem)
        cp.start()
        cp.wait()
        if pad_rows > 0:
            x_ref[pl.ds(hi_real, pad_rows), pl.ds(cc, cc)] = jnp.full(
                (pad_rows, cc), jnp.inf, dtype=jnp.float32)

        r_iota = jax.lax.broadcasted_iota(jnp.int32, (blk, 1), 0)
        lane = jax.lax.broadcasted_iota(jnp.int32, (1, 2 * cc), 1)
        lane_hi = jnp.where(lane >= cc, jnp.int32(rows), jnp.int32(0))
        bs_tab = [(r_iota >> s) & 1 for s in range(n_small)]

        def prologue_block(g, _):
            xb = x_ref[pl.ds(g * blk, blk), :]
            base_r = g * blk + r_iota
            for k in range(1, n_small):
                bk = (base_r >> k) & 1
                for s in range(k - 1, -1, -1):
                    dd = 1 << s
                    up = pltpu.roll(xb, blk - dd, axis=0)
                    dn = pltpu.roll(xb, dd, axis=0)
                    bs = bs_tab[s]
                    p = jnp.where(bs == 0, up, dn)
                    take = (xb > p) == (bs == bk)
                    xb = jnp.where(take, p, xb)
            x_ref[pl.ds(g * blk, blk), :] = xb
            return 0

        jax.lax.fori_loop(0, n_blocks, prologue_block, 0)

        def lane_swap_block(g, _):
            xb = x_ref[pl.ds(g * blk, blk), :]
            lo = xb[:, :cc]
            hi = xb[:, cc:]
            x_ref[pl.ds(g * blk, blk), :] = jnp.concatenate(
                [jnp.minimum(lo, hi), jnp.maximum(lo, hi)], axis=1)
            return 0

        def emit_phase(k, static):
            if static and k == log_n:
                jax.lax.fori_loop(0, n_blocks, lane_swap_block, 0)

            def asc_of(base):
                if static:
                    return (((base + lane_hi) >> k) & 1) == 0
                return ((base >> k) & 1) == 0

            for s in range(s_swap - 1, log_b - 1, -1):
                d = 1 << s
                db = d >> log_b

                def pair(g, _, d=d, db=db):
                    base = ((g // db) * 2 * db + g % db) * blk
                    a = x_ref[pl.ds(base, blk), :]
                    b = x_ref[pl.ds(base + d, blk), :]
                    mn = jnp.minimum(a, b)
                    mx = jnp.maximum(a, b)
                    asc = asc_of(base)
                    x_ref[pl.ds(base, blk), :] = jnp.where(asc, mn, mx)
                    x_ref[pl.ds(base + d, blk), :] = jnp.where(asc, mx, mn)
                    return 0

                if static:
                    jax.lax.fori_loop(0, n_blocks // 2, pair, 0)
                else:
                    @pl.when(k > s)
                    def _(pair=pair):
                        jax.lax.fori_loop(0, n_blocks // 2, pair, 0)

            for s in range(log_b - 1, n_small - 1, -1):
                d = 1 << s
                trips = rows // (2 * d)
                unroll = next(u for u in (4, 2, 1) if trips % u == 0)

                def one_pair(base, d=d):
                    a = x_ref[pl.ds(base, d), :]
                    b = x_ref[pl.ds(base + d, d), :]
                    mn = jnp.minimum(a, b)
                    mx = jnp.maximum(a, b)
                    asc = asc_of(base)
                    x_ref[pl.ds(base, d), :] = jnp.where(asc, mn, mx)
                    x_ref[pl.ds(base + d, d), :] = jnp.where(asc, mx, mn)

                def pairm(i, _, d=d, unroll=unroll):
                    for u in range(unroll):
                        one_pair((i * unroll + u) * 2 * d)
                    return 0

                if static:
                    jax.lax.fori_loop(0, trips // unroll, pairm, 0)
                else:
                    @pl.when(k > s)
                    def _(pairm=pairm, trips=trips, unroll=unroll):
                        jax.lax.fori_loop(0, trips // unroll, pairm, 0)

            def small_block(g, _):
                xb = x_ref[pl.ds(g * blk, blk), :]
                if static:
                    bk = ((g * blk + r_iota + lane_hi) >> k) & 1
                else:
                    bk = ((g * blk + r_iota) >> k) & 1
                for s in range(n_small - 1, -1, -1):
                    dd = 1 << s
                    up = pltpu.roll(xb, blk - dd, axis=0)
                    dn = pltpu.roll(xb, dd, axis=0)
                    bs = bs_tab[s]
                    p = jnp.where(bs == 0, up, dn)
                    wm = bs == bk
                    take = (xb > p) == wm
                    xb = jnp.where(take, p, xb)
                x_ref[pl.ds(g * blk, blk), :] = xb
                return 0

            jax.lax.fori_loop(0, n_blocks, small_block, 0)

        def phase(k, _):
            emit_phase(k, static=False)
            return 0

        if n_small < s_swap:
            jax.lax.fori_loop(n_small, s_swap, phase, 0)
        for k in range(max(s_swap, n_small), log_n + 1):
            emit_phase(k, static=True)

        def red_block(g, acc):
            xb = x_ref[pl.ds(g * blk, blk), :]
            dl = xb[:, 0:half] - xb[:, half:cc]
            dh = xb[:, cc:cc + half] - xb[:, cc + half:]
            valid = (r_iota + g * blk) < hi_real
            dh = jnp.where(valid, dh, 0.0)
            return acc + jnp.sum(dl * dl) + jnp.sum(dh * dh)

        acc = jax.lax.fori_loop(0, n_blocks, red_block, jnp.float32(0.0))

        @pl.when(c == 0)
        def _():
            o_ref[...] = jnp.zeros((1, 1), jnp.float32)

        o_ref[...] += acc.reshape(1, 1)

    return body


def kernel(XP, XQ, projections):
    n, f = XP.shape
    l = projections.shape[1]
    n_pad = 1 << (n - 1).bit_length()
    rows = n_pad // 2

    cc = min(64, 2 * l)
    half = cc // 2
    n_chunks = (2 * l) // cc
    blk = min(4096, rows)

    P2 = jnp.zeros((2 * f, 2 * l), dtype=jnp.float32)
    P2 = P2.at[:f, :l].set(projections).at[f:, l:].set(projections)
    perm = []
    for g in range(n_chunks):
        perm += list(range(g * half, (g + 1) * half))
        perm += list(range(l + g * half, l + (g + 1) * half))
    P2 = P2[:, np.array(perm)]
    P3 = P2.reshape(2 * f, n_chunks, cc).transpose(1, 0, 2)

    X2 = jnp.zeros((n_pad, 2 * f), dtype=jnp.float32)
    X2 = X2.at[:n, :f].set(XP).at[:n, f:].set(XQ)

    rb = min(4096, rows)
    n_rb = rows // rb
    Z = pl.pallas_call(
        _matmul_body,
        grid=(n_rb, n_chunks),
        in_specs=[
            pl.BlockSpec((rb, 2 * f), lambda r, c: (r, 0)),
            pl.BlockSpec((rb, 2 * f), lambda r, c, n_rb=n_rb: (r + n_rb, 0)),
            pl.BlockSpec((1, 2 * f, cc), lambda r, c: (c, 0, 0)),
        ],
        out_specs=pl.BlockSpec((1, rb, 2 * cc), lambda r, c: (c, r, 0)),
        out_shape=jax.ShapeDtypeStruct((n_chunks, rows, 2 * cc), jnp.float32),
    )(X2, X2, P3)

    total = pl.pallas_call(
        _make_sort_body(n, n_pad, cc, blk),
        grid=(n_chunks,),
        in_specs=[pl.BlockSpec(memory_space=pltpu.MemorySpace.HBM)],
        out_specs=pl.BlockSpec((1, 1), lambda c: (0, 0)),
        out_shape=jax.ShapeDtypeStruct((1, 1), jnp.float32),
        scratch_shapes=[
            pltpu.VMEM((rows, 2 * cc), jnp.float32),
            pltpu.SemaphoreType.DMA,
        ],
    )(Z)

    return jnp.sqrt(total[0, 0] / (n * l))


# speedup vs baseline: 1.0319x; 1.0093x over previous
import numpy as np

import jax
import jax.numpy as jnp
from jax.experimental import pallas as pl
from jax.experimental.pallas import tpu as pltpu


def _matmul_body(x1_ref, x2_ref, p_ref, o_ref):
    p = p_ref[0]
    a = jnp.dot(x1_ref[...], p, preferred_element_type=jnp.float32)
    b = jnp.dot(x2_ref[...], p, preferred_element_type=jnp.float32)
    o_ref[...] = jnp.concatenate([a, b], axis=1)[None]


def _make_sort_body(n, n_pad, cc, blk):
    rows = n_pad // 2
    log_n = n_pad.bit_length() - 1
    s_swap = rows.bit_length() - 1
    log_b = blk.bit_length() - 1
    n_blocks = rows // blk
    hi_real = n - rows
    pad_rows = rows - hi_real
    half = cc // 2
    n_small = min(5, log_b)

    def body(z_ref, o_ref, x_ref, sem):
        c = pl.program_id(0)
        cp = pltpu.make_async_copy(z_ref.at[c], x_ref, sem)
        cp.start()
        cp.wait()
        if pad_rows > 0:
            x_ref[pl.ds(hi_real, pad_rows), pl.ds(cc, cc)] = jnp.full(
                (pad_rows, cc), jnp.inf, dtype=jnp.float32)

        r_iota = jax.lax.broadcasted_iota(jnp.int32, (blk, 1), 0)
        lane = jax.lax.broadcasted_iota(jnp.int32, (1, 2 * cc), 1)
        lane_hi = jnp.where(lane >= cc, jnp.int32(rows), jnp.int32(0))
        bs_tab = [(r_iota >> s) & 1 for s in range(n_small)]

        def prologue_block(g, _):
            xb = x_ref[pl.ds(g * blk, blk), :]
            base_r = g * blk + r_iota
            for k in range(1, n_small):
                bk = (base_r >> k) & 1
                for s in range(k - 1, -1, -1):
                    dd = 1 << s
                    up = pltpu.roll(xb, blk - dd, axis=0)
                    dn = pltpu.roll(xb, dd, axis=0)
                    bs = bs_tab[s]
                    p = jnp.where(bs == 0, up, dn)
                    take = (xb > p) == (bs == bk)
                    xb = jnp.where(take, p, xb)
            x_ref[pl.ds(g * blk, blk), :] = xb
            return 0

        jax.lax.fori_loop(0, n_blocks, prologue_block, 0)

        def lane_swap_block(g, _):
            xb = x_ref[pl.ds(g * blk, blk), :]
            lo = xb[:, :cc]
            hi = xb[:, cc:]
            x_ref[pl.ds(g * blk, blk), :] = jnp.concatenate(
                [jnp.minimum(lo, hi), jnp.maximum(lo, hi)], axis=1)
            return 0

        def emit_phase(k, static):
            if static and k == log_n:
                jax.lax.fori_loop(0, n_blocks, lane_swap_block, 0)

            def asc_of(base):
                if static:
                    return (((base + lane_hi) >> k) & 1) == 0
                return ((base >> k) & 1) == 0

            for s in range(s_swap - 1, log_b - 1, -1):
                d = 1 << s
                db = d >> log_b

                def pair(g, _, d=d, db=db):
                    base = ((g // db) * 2 * db + g % db) * blk
                    a = x_ref[pl.ds(base, blk), :]
                    b = x_ref[pl.ds(base + d, blk), :]
                    mn = jnp.minimum(a, b)
                    mx = jnp.maximum(a, b)
                    asc = asc_of(base)
                    x_ref[pl.ds(base, blk), :] = jnp.where(asc, mn, mx)
                    x_ref[pl.ds(base + d, blk), :] = jnp.where(asc, mx, mn)
                    return 0

                if static:
                    jax.lax.fori_loop(0, n_blocks // 2, pair, 0)
                else:
                    @pl.when(k > s)
                    def _(pair=pair):
                        jax.lax.fori_loop(0, n_blocks // 2, pair, 0)

            for s in range(log_b - 1, n_small - 1, -1):
                d = 1 << s
                trips = rows // (2 * d)
                unroll = next(u for u in (8, 4, 2, 1) if trips % u == 0)

                def one_pair(base, d=d):
                    a = x_ref[pl.ds(base, d), :]
                    b = x_ref[pl.ds(base + d, d), :]
                    mn = jnp.minimum(a, b)
                    mx = jnp.maximum(a, b)
                    asc = asc_of(base)
                    x_ref[pl.ds(base, d), :] = jnp.where(asc, mn, mx)
                    x_ref[pl.ds(base + d, d), :] = jnp.where(asc, mx, mn)

                def pairm(i, _, d=d, unroll=unroll):
                    for u in range(unroll):
                        one_pair((i * unroll + u) * 2 * d)
                    return 0

                if static:
                    jax.lax.fori_loop(0, trips // unroll, pairm, 0)
                else:
                    @pl.when(k > s)
                    def _(pairm=pairm, trips=trips, unroll=unroll):
                        jax.lax.fori_loop(0, trips // unroll, pairm, 0)

            def small_block(g, _):
                xb = x_ref[pl.ds(g * blk, blk), :]
                if static:
                    bk = ((g * blk + r_iota + lane_hi) >> k) & 1
                else:
                    bk = ((g * blk + r_iota) >> k) & 1
                for s in range(n_small - 1, -1, -1):
                    dd = 1 << s
                    up = pltpu.roll(xb, blk - dd, axis=0)
                    dn = pltpu.roll(xb, dd, axis=0)
                    bs = bs_tab[s]
                    p = jnp.where(bs == 0, up, dn)
                    wm = bs == bk
                    take = (xb > p) == wm
                    xb = jnp.where(take, p, xb)
                x_ref[pl.ds(g * blk, blk), :] = xb
                return 0

            jax.lax.fori_loop(0, n_blocks, small_block, 0)

        def phase(k, _):
            emit_phase(k, static=False)
            return 0

        if n_small < s_swap:
            jax.lax.fori_loop(n_small, s_swap, phase, 0)
        for k in range(max(s_swap, n_small), log_n + 1):
            emit_phase(k, static=True)

        def red_block(g, acc):
            xb = x_ref[pl.ds(g * blk, blk), :]
            dl = xb[:, 0:half] - xb[:, half:cc]
            dh = xb[:, cc:cc + half] - xb[:, cc + half:]
            valid = (r_iota + g * blk) < hi_real
            dh = jnp.where(valid, dh, 0.0)
            return acc + jnp.sum(dl * dl) + jnp.sum(dh * dh)

        acc = jax.lax.fori_loop(0, n_blocks, red_block, jnp.float32(0.0))

        @pl.when(c == 0)
        def _():
            o_ref[...] = jnp.zeros((1, 1), jnp.float32)

        o_ref[...] += acc.reshape(1, 1)

    return body


def kernel(XP, XQ, projections):
    n, f = XP.shape
    l = projections.shape[1]
    n_pad = 1 << (n - 1).bit_length()
    rows = n_pad // 2

    cc = min(64, 2 * l)
    half = cc // 2
    n_chunks = (2 * l) // cc
    blk = min(4096, rows)

    P2 = jnp.zeros((2 * f, 2 * l), dtype=jnp.float32)
    P2 = P2.at[:f, :l].set(projections).at[f:, l:].set(projections)
    perm = []
    for g in range(n_chunks):
        perm += list(range(g * half, (g + 1) * half))
        perm += list(range(l + g * half, l + (g + 1) * half))
    P2 = P2[:, np.array(perm)]
    P3 = P2.reshape(2 * f, n_chunks, cc).transpose(1, 0, 2)

    X2 = jnp.zeros((n_pad, 2 * f), dtype=jnp.float32)
    X2 = X2.at[:n, :f].set(XP).at[:n, f:].set(XQ)

    rb = min(4096, rows)
    n_rb = rows // rb
    Z = pl.pallas_call(
        _matmul_body,
        grid=(n_rb, n_chunks),
        in_specs=[
            pl.BlockSpec((rb, 2 * f), lambda r, c: (r, 0)),
            pl.BlockSpec((rb, 2 * f), lambda r, c, n_rb=n_rb: (r + n_rb, 0)),
            pl.BlockSpec((1, 2 * f, cc), lambda r, c: (c, 0, 0)),
        ],
        out_specs=pl.BlockSpec((1, rb, 2 * cc), lambda r, c: (c, r, 0)),
        out_shape=jax.ShapeDtypeStruct((n_chunks, rows, 2 * cc), jnp.float32),
    )(X2, X2, P3)

    total = pl.pallas_call(
        _make_sort_body(n, n_pad, cc, blk),
        grid=(n_chunks,),
        in_specs=[pl.BlockSpec(memory_space=pltpu.MemorySpace.HBM)],
        out_specs=pl.BlockSpec((1, 1), lambda c: (0, 0)),
        out_shape=jax.ShapeDtypeStruct((1, 1), jnp.float32),
        scratch_shapes=[
            pltpu.VMEM((rows, 2 * cc), jnp.float32),
            pltpu.SemaphoreType.DMA,
        ],
    )(Z)

    return jnp.sqrt(total[0, 0] / (n * l))
